# baseline (speedup 1.0000x reference)
import numpy as np
import jax
import jax.numpy as jnp
from jax.experimental import pallas as pl
from jax.experimental.pallas import tpu as pltpu

_BN_EPS = 1e-5

_B, _C, _H, _W = 16, 256, 64, 64
_T = _H * _W
_TA = _T // 4
_CHUNK = 512
_NCHUNK = _TA // _CHUNK
_OUT_C = 512


def _spa_full_idx() -> np.ndarray:
    idx = np.arange(_T)
    a_idx = idx[::4]
    b_idx = idx[idx % 4 != 0]
    width = int(np.sqrt(_T))
    ac = np.stack([a_idx // width, a_idx % width], -1).astype(np.float32)
    bc = np.stack([b_idx // width, b_idx % width], -1).astype(np.float32)
    dist = np.sqrt(((ac[:, None, :] - bc[None, :, :]) ** 2).sum(-1))
    return b_idx[np.argmax(1.0 / (dist + 1e-6), axis=-1)]


_SPA_FULL = _spa_full_idx()


def _body(x_ref, xa_ref, xspa_ref, w_ref, g_ref, be_ref, mu_ref, va_ref,
          fw_ref, o_ref):
    xf = x_ref[0]
    xa = xa_ref[0]
    xspa = xspa_ref[0]

    an = xa / jnp.sqrt(jnp.sum(xa * xa, axis=0, keepdims=True))
    bn = xf / jnp.sqrt(jnp.sum(xf * xf, axis=0, keepdims=True))

    simT = jax.lax.dot_general(bn, an, (((0,), (0,)), ((), ())),
                               preferred_element_type=jnp.float32)

    r8 = jax.lax.broadcasted_iota(jnp.int32, (8, 128), 0)
    pen8 = jnp.where((r8 & 3) == 0, -3e38, 0.0).astype(jnp.float32)
    pen = pltpu.repeat(pltpu.repeat(pen8, _T // 8, 0), _CHUNK // 128, 1)
    simT = simT + pen

    m = jnp.max(simT, axis=0, keepdims=True)
    onehot = jnp.where(simT == m, 1.0, 0.0)

    fw = jnp.clip(fw_ref[...], 0.0, 6.0)
    fwn = fw / (jnp.sum(fw) + 1e-8)
    csim = 0.5 * fwn[0, 0]
    cspa = 0.5 * fwn[0, 1]

    sel = jax.lax.dot_general(xf, onehot, (((1,), (0,)), ((), ())),
                              preferred_element_type=jnp.float32)
    fused = (csim + cspa) * xa + cspa * xspa + csim * sel

    out = jax.lax.dot_general(w_ref[...], fused, (((1,), (0,)), ((), ())),
                              preferred_element_type=jnp.float32)
    scale = g_ref[...] / jnp.sqrt(va_ref[...] + _BN_EPS)
    bias = be_ref[...] - mu_ref[...] * scale
    y = out * scale + bias
    o_ref[0] = y * jax.nn.sigmoid(y)


def kernel(x, conv_w, bn_gamma, bn_beta, bn_mean, bn_var, fusion_weights):
    B, C, H, W = x.shape
    xr = x.reshape(B, C, _T)
    xa = xr[:, :, ::4]
    xspa = xr[:, :, _SPA_FULL]

    grid = (B, _NCHUNK)
    out = pl.pallas_call(
        _body,
        grid=grid,
        in_specs=[
            pl.BlockSpec((1, C, _T), lambda b, i: (b, 0, 0)),
            pl.BlockSpec((1, C, _CHUNK), lambda b, i: (b, 0, i)),
            pl.BlockSpec((1, C, _CHUNK), lambda b, i: (b, 0, i)),
            pl.BlockSpec((_OUT_C, C), lambda b, i: (0, 0)),
            pl.BlockSpec((_OUT_C, 1), lambda b, i: (0, 0)),
            pl.BlockSpec((_OUT_C, 1), lambda b, i: (0, 0)),
            pl.BlockSpec((_OUT_C, 1), lambda b, i: (0, 0)),
            pl.BlockSpec((_OUT_C, 1), lambda b, i: (0, 0)),
            pl.BlockSpec((1, 2), lambda b, i: (0, 0)),
        ],
        out_specs=pl.BlockSpec((1, _OUT_C, _CHUNK), lambda b, i: (b, 0, i)),
        out_shape=jax.ShapeDtypeStruct((B, _OUT_C, _TA), jnp.float32),
        compiler_params=pltpu.CompilerParams(
            dimension_semantics=("parallel", "arbitrary"),
            vmem_limit_bytes=100 * 1024 * 1024,
        ),
    )(
        xr, xa, xspa, conv_w,
        bn_gamma.reshape(_OUT_C, 1), bn_beta.reshape(_OUT_C, 1),
        bn_mean.reshape(_OUT_C, 1), bn_var.reshape(_OUT_C, 1),
        fusion_weights.reshape(1, 2),
    )
    return out.reshape(B, _OUT_C, H // 2, W // 2)


# speedup vs baseline: 2.4563x; 2.4563x over previous
import numpy as np
import jax
import jax.numpy as jnp
from jax.experimental import pallas as pl
from jax.experimental.pallas import tpu as pltpu

_BN_EPS = 1e-5

_B, _C, _H, _W = 16, 256, 64, 64
_T = _H * _W
_TA = _T // 4
_CHUNK = 512
_NCHUNK = _TA // _CHUNK
_OUT_C = 512


def _spa_full_idx() -> np.ndarray:
    idx = np.arange(_T)
    a_idx = idx[::4]
    b_idx = idx[idx % 4 != 0]
    width = int(np.sqrt(_T))
    ac = np.stack([a_idx // width, a_idx % width], -1).astype(np.float32)
    bc = np.stack([b_idx // width, b_idx % width], -1).astype(np.float32)
    dist = np.sqrt(((ac[:, None, :] - bc[None, :, :]) ** 2).sum(-1))
    return b_idx[np.argmax(1.0 / (dist + 1e-6), axis=-1)]


_SPA_FULL = _spa_full_idx()


def _pa_mat() -> np.ndarray:
    p = np.zeros((_NCHUNK, _T, _CHUNK), np.float32)
    for c in range(_NCHUNK):
        cols = np.arange(_CHUNK)
        p[c, 4 * (c * _CHUNK + cols), cols] = 1.0
    return p


_PA = _pa_mat()


def _body(x_ref, pa_ref, xspa_ref, w_ref, g_ref, be_ref, mu_ref, va_ref,
          fw_ref, o_ref):
    xf = x_ref[0]
    xspa = xspa_ref[0]

    xa = jax.lax.dot_general(xf, pa_ref[0], (((1,), (0,)), ((), ())),
                             preferred_element_type=jnp.float32)

    an = xa / jnp.sqrt(jnp.sum(xa * xa, axis=0, keepdims=True))
    bn = xf / jnp.sqrt(jnp.sum(xf * xf, axis=0, keepdims=True))

    simT = jax.lax.dot_general(bn, an, (((0,), (0,)), ((), ())),
                               preferred_element_type=jnp.float32)

    r8 = jax.lax.broadcasted_iota(jnp.int32, (8, 128), 0)
    pen8 = jnp.where((r8 & 3) == 0, -3e38, 0.0).astype(jnp.float32)
    pen = pltpu.repeat(pltpu.repeat(pen8, _T // 8, 0), _CHUNK // 128, 1)
    simT = simT + pen

    m = jnp.max(simT, axis=0, keepdims=True)
    onehot = jnp.where(simT == m, 1.0, 0.0)

    fw = jnp.clip(fw_ref[...], 0.0, 6.0)
    fwn = fw / (jnp.sum(fw) + 1e-8)
    csim = 0.5 * fwn[0, 0]
    cspa = 0.5 * fwn[0, 1]

    sel = jax.lax.dot_general(xf, onehot, (((1,), (0,)), ((), ())),
                              preferred_element_type=jnp.float32)
    fused = (csim + cspa) * xa + cspa * xspa + csim * sel

    out = jax.lax.dot_general(w_ref[...], fused, (((1,), (0,)), ((), ())),
                              preferred_element_type=jnp.float32)
    scale = g_ref[...] / jnp.sqrt(va_ref[...] + _BN_EPS)
    bias = be_ref[...] - mu_ref[...] * scale
    y = out * scale + bias
    o_ref[0] = y * jax.nn.sigmoid(y)


def kernel(x, conv_w, bn_gamma, bn_beta, bn_mean, bn_var, fusion_weights):
    B, C, H, W = x.shape
    xr = x.reshape(B, C, _T)
    xspa = xr[:, :, _SPA_FULL]

    grid = (_NCHUNK, B)
    out = pl.pallas_call(
        _body,
        grid=grid,
        in_specs=[
            pl.BlockSpec((1, C, _T), lambda i, b: (b, 0, 0)),
            pl.BlockSpec((1, _T, _CHUNK), lambda i, b: (i, 0, 0)),
            pl.BlockSpec((1, C, _CHUNK), lambda i, b: (b, 0, i)),
            pl.BlockSpec((_OUT_C, C), lambda i, b: (0, 0)),
            pl.BlockSpec((_OUT_C, 1), lambda i, b: (0, 0)),
            pl.BlockSpec((_OUT_C, 1), lambda i, b: (0, 0)),
            pl.BlockSpec((_OUT_C, 1), lambda i, b: (0, 0)),
            pl.BlockSpec((_OUT_C, 1), lambda i, b: (0, 0)),
            pl.BlockSpec((1, 2), lambda i, b: (0, 0)),
        ],
        out_specs=pl.BlockSpec((1, _OUT_C, _CHUNK), lambda i, b: (b, 0, i)),
        out_shape=jax.ShapeDtypeStruct((B, _OUT_C, _TA), jnp.float32),
        compiler_params=pltpu.CompilerParams(
            dimension_semantics=("parallel", "arbitrary"),
            vmem_limit_bytes=100 * 1024 * 1024,
        ),
    )(
        xr, _PA, xspa, conv_w,
        bn_gamma.reshape(_OUT_C, 1), bn_beta.reshape(_OUT_C, 1),
        bn_mean.reshape(_OUT_C, 1), bn_var.reshape(_OUT_C, 1),
        fusion_weights.reshape(1, 2),
    )
    return out.reshape(B, _OUT_C, H // 2, W // 2)


# speedup vs baseline: 2.7092x; 1.1030x over previous
import numpy as np
import jax
import jax.numpy as jnp
from jax.experimental import pallas as pl
from jax.experimental.pallas import tpu as pltpu

_BN_EPS = 1e-5

_B, _C, _H, _W = 16, 256, 64, 64
_T = _H * _W
_TA = _T // 4
_CHUNK = 512
_NCHUNK = _TA // _CHUNK
_OUT_C = 512


def _spa_full_idx() -> np.ndarray:
    idx = np.arange(_T)
    a_idx = idx[::4]
    b_idx = idx[idx % 4 != 0]
    width = int(np.sqrt(_T))
    ac = np.stack([a_idx // width, a_idx % width], -1).astype(np.float32)
    bc = np.stack([b_idx // width, b_idx % width], -1).astype(np.float32)
    dist = np.sqrt(((ac[:, None, :] - bc[None, :, :]) ** 2).sum(-1))
    return b_idx[np.argmax(1.0 / (dist + 1e-6), axis=-1)]


_IDX2 = np.concatenate([np.arange(0, _T, 4), _spa_full_idx()])


def _body(x_ref, xas_ref, w_ref, g_ref, be_ref, mu_ref, va_ref,
          fw_ref, o_ref):
    xf = x_ref[0]
    xas = xas_ref[0]

    bn = xf / jnp.sqrt(jnp.sum(xf * xf, axis=0, keepdims=True))

    fw = jnp.clip(fw_ref[...], 0.0, 6.0)
    fwn = fw / (jnp.sum(fw) + 1e-8)
    csim = 0.5 * fwn[0, 0]
    cspa = 0.5 * fwn[0, 1]

    scale = g_ref[...] / jnp.sqrt(va_ref[...] + _BN_EPS)
    bias = be_ref[...] - mu_ref[...] * scale

    r8 = jax.lax.broadcasted_iota(jnp.int32, (8, 128), 0)
    pen8 = jnp.where((r8 & 3) == 0, -3e38, 0.0).astype(jnp.float32)
    pen = pltpu.repeat(pltpu.repeat(pen8, _T // 8, 0), _CHUNK // 128, 1)

    for c in range(_NCHUNK):
        lo = c * _CHUNK
        xa = xas[:, lo:lo + _CHUNK]
        xspa = xas[:, _TA + lo:_TA + lo + _CHUNK]

        an = xa / jnp.sqrt(jnp.sum(xa * xa, axis=0, keepdims=True))

        simT = jax.lax.dot_general(bn, an, (((0,), (0,)), ((), ())),
                                   preferred_element_type=jnp.float32)
        simT = simT + pen

        m = jnp.max(simT, axis=0, keepdims=True)
        onehot = jnp.where(simT == m, 1.0, 0.0)

        sel = jax.lax.dot_general(xf, onehot, (((1,), (0,)), ((), ())),
                                  preferred_element_type=jnp.float32)
        fused = (csim + cspa) * xa + cspa * xspa + csim * sel

        out = jax.lax.dot_general(w_ref[...], fused, (((1,), (0,)), ((), ())),
                                  preferred_element_type=jnp.float32)
        y = out * scale + bias
        o_ref[0, :, lo:lo + _CHUNK] = y * jax.nn.sigmoid(y)


def kernel(x, conv_w, bn_gamma, bn_beta, bn_mean, bn_var, fusion_weights):
    B, C, H, W = x.shape
    xr = x.reshape(B, C, _T)
    xas = xr[:, :, _IDX2]

    grid = (B,)
    out = pl.pallas_call(
        _body,
        grid=grid,
        in_specs=[
            pl.BlockSpec((1, C, _T), lambda b: (b, 0, 0)),
            pl.BlockSpec((1, C, 2 * _TA), lambda b: (b, 0, 0)),
            pl.BlockSpec((_OUT_C, C), lambda b: (0, 0)),
            pl.BlockSpec((_OUT_C, 1), lambda b: (0, 0)),
            pl.BlockSpec((_OUT_C, 1), lambda b: (0, 0)),
            pl.BlockSpec((_OUT_C, 1), lambda b: (0, 0)),
            pl.BlockSpec((_OUT_C, 1), lambda b: (0, 0)),
            pl.BlockSpec((1, 2), lambda b: (0, 0)),
        ],
        out_specs=pl.BlockSpec((1, _OUT_C, _TA), lambda b: (b, 0, 0)),
        out_shape=jax.ShapeDtypeStruct((B, _OUT_C, _TA), jnp.float32),
        compiler_params=pltpu.CompilerParams(
            dimension_semantics=("parallel",),
            vmem_limit_bytes=100 * 1024 * 1024,
        ),
    )(
        xr, xas, conv_w,
        bn_gamma.reshape(_OUT_C, 1), bn_beta.reshape(_OUT_C, 1),
        bn_mean.reshape(_OUT_C, 1), bn_var.reshape(_OUT_C, 1),
        fusion_weights.reshape(1, 2),
    )
    return out.reshape(B, _OUT_C, H // 2, W // 2)
